# Initial kernel scaffold; baseline (speedup 1.0000x reference)
import jax
import jax.numpy as jnp
from jax.experimental import pallas as pl


def kernel(uids, u_item_pad, u_user_pad, u_user_item_pad, user_table, item_table, rating_table, gv_W1, gv_b1, gv_W2, gv_b2, ui_W1, ui_b1, ui_W2, ui_b2, ai_W, ai_b, uu_W1, uu_b1, uu_W2, uu_b2, an_W, an_b, m_W1, m_b1, m_W2, m_b2, m_W3, m_b3):
    raise NotImplementedError("write your pallas kernel here")


# speedup vs baseline: 3.0573x; 3.0573x over previous
import functools

import jax
import jax.numpy as jnp
from jax import lax
from jax.experimental import pallas as pl
from jax.experimental.pallas import tpu as pltpu
from jax.experimental.pallas import tpu_sc as plsc

D = 64
EPS = 1e-10
NW = 32
CHUNK = 128
FIRE = 7


def _sc_gather(item_table, rating_table, user_table, item_idx, rating_idx,
               uid_idx, nbr_idx):
    item_chunks = item_idx.shape[0] // (NW * CHUNK)
    nbr_chunks = nbr_idx.shape[0] // (NW * CHUNK)
    uid_per = uid_idx.shape[0] // NW

    mesh = plsc.VectorSubcoreMesh(core_axis_name="c", subcore_axis_name="s")

    @functools.partial(
        pl.kernel,
        out_type=[
            jax.ShapeDtypeStruct((item_chunks * NW, CHUNK, D), jnp.float32),
            jax.ShapeDtypeStruct((item_chunks * NW, CHUNK, D), jnp.float32),
            jax.ShapeDtypeStruct((uid_idx.shape[0], D), jnp.float32),
            jax.ShapeDtypeStruct((nbr_chunks * NW, CHUNK, D), jnp.float32),
        ],
        mesh=mesh,
        scratch_types=[
            pltpu.VMEM((item_chunks * CHUNK,), jnp.int32),
            pltpu.VMEM((FIRE, CHUNK, D), jnp.float32),
            pltpu.VMEM((uid_per,), jnp.int32),
            pltpu.VMEM((uid_per, D), jnp.float32),
            pltpu.SemaphoreType.DMA,
        ],
        compiler_params=pltpu.CompilerParams(use_tc_tiling_on_sc=False),
    )
    def gather_k(item_t, rating_t, user_t, item_i, rating_i, uid_i, nbr_i,
                 item_o, rating_o, uid_o, nbr_o,
                 idx_v, rows_v, uidx_v, urows_v, sem):
        wid = lax.axis_index("s") * 2 + lax.axis_index("c")

        def job(tab, idx_h, out_h, n_chunks):
            base = wid * n_chunks
            per_tile = n_chunks * CHUNK
            pltpu.sync_copy(idx_h.at[pl.ds(wid * per_tile, per_tile)],
                            idx_v.at[pl.ds(0, per_tile)])
            n_groups = n_chunks // FIRE
            rem = n_chunks - n_groups * FIRE

            def fire_drain(g, k):
                cps = [
                    pltpu.make_async_copy(
                        tab.at[idx_v.at[pl.ds((g * FIRE + j) * CHUNK, CHUNK)]],
                        rows_v.at[j], sem)
                    for j in range(k)
                ]
                for c in cps:
                    c.start()
                for c in cps:
                    c.wait()
                pltpu.sync_copy(rows_v.at[pl.ds(0, k)],
                                out_h.at[pl.ds(base + g * FIRE, k)])

            def body(g, carry):
                fire_drain(g, FIRE)
                return carry

            lax.fori_loop(0, n_groups, body, 0)
            if rem:
                fire_drain(n_groups, rem)

        job(item_t, item_i, item_o, item_chunks)
        job(rating_t, rating_i, rating_o, item_chunks)
        job(user_t, nbr_i, nbr_o, nbr_chunks)
        pltpu.sync_copy(uid_i.at[pl.ds(wid * uid_per, uid_per)], uidx_v)
        pltpu.async_copy(user_t.at[uidx_v], urows_v, sem).wait()
        pltpu.sync_copy(urows_v, uid_o.at[pl.ds(wid * uid_per, uid_per)])

    return gather_k(item_table, rating_table, user_table, item_idx,
                    rating_idx, uid_idx, nbr_idx)


def _seg_mat(per, rows, cols, dtype):
    c = lax.broadcasted_iota(jnp.int32, (rows, cols), 1)
    r = lax.broadcasted_iota(jnp.int32, (rows, cols), 0)
    return (c // per == r).astype(dtype)


def _rep_mat(per, rows, cols, dtype):
    c = lax.broadcasted_iota(jnp.int32, (rows, cols), 1)
    r = lax.broadcasted_iota(jnp.int32, (rows, cols), 0)
    return (r // per == c).astype(dtype)


def _tc_compute(BU, B, Li, Nn, H,
                item_g, rating_g, pu_g, nbr_g, mk1_i, mkE_i, mkO_i, mku_i,
                gvA, gvB, gvb1, gvW2t, gvb2, uiA, uiB, uib1, uiw2, uib2,
                aiWt, aib, uuA, uuB, uub1, uuw2, uub2, anWt, anb,
                m1A, m1B, mb1, mW2t, mb2, mW3t, mb3):
    G = B // BU
    R1 = BU * Li
    R2 = BU * Nn * H
    RN = BU * Nn
    EQ_OFF = (B * Li) // R2
    OQ_OFF = EQ_OFF + (B * Nn * H) // R2
    f32 = jnp.float32

    def dot(a, b):
        return lax.dot_general(a, b, (((1,), (0,)), ((), ())),
                               preferred_element_type=f32)

    def body(qa, ra, eq, oq, eer, oer, pu, nbr, mk1i, mkEi, mkOi, mkui,
             gvA_r, gvB_r, gvb1_r, gvW2t_r, gvb2_r, uiA_r, uiB_r, uib1_r,
             uiw2_r, uib2_r, aiWt_r, aib_r, uuA_r, uuB_r, uub1_r, uuw2_r,
             uub2_r, anWt_r, anb_r, m1A_r, m1B_r, mb1_r, mW2t_r, mb2_r,
             mW3t_r, mb3_r, out):
        relu = lambda x: jnp.maximum(x, 0.0)
        gvA_, gvB_, gvb1_ = gvA_r[...], gvB_r[...], gvb1_r[...]
        gvW2t_, gvb2_ = gvW2t_r[...], gvb2_r[...]
        uiA_, uiB_, uib1_ = uiA_r[...], uiB_r[...], uib1_r[...]
        uiw2_, uib2_ = uiw2_r[...], uib2_r[...]
        aiWt_, aib_ = aiWt_r[...], aib_r[...]

        seg50 = _seg_mat(Li, BU, R1, f32)
        rep50 = _rep_mat(Li, R1, BU, f32)
        seg10 = _seg_mat(H, RN, R2, f32)
        rep10 = _rep_mat(H, R2, RN, f32)
        seg20 = _seg_mat(Nn, BU, RN, f32)

        mk1 = (mk1i[...] > 0).astype(f32)
        mkE = (mkEi[...] > 0).astype(f32)
        mkO = (mkOi[...] > 0).astype(f32)
        mku = (mkui[...] > 0).astype(f32)
        pu_ = pu[...]
        nbr_ = nbr[...]

        xh = relu(dot(qa[...], gvA_) + dot(ra[...], gvB_) + gvb1_)
        x_ia = dot(xh, gvW2t_) + gvb2_
        puproj = dot(pu_, uiB_)
        t = relu(dot(x_ia, uiA_) + mk1 * dot(rep50, puproj) + uib1_)
        logit = jnp.sum(t * uiw2_, axis=1, keepdims=True) + uib2_
        a = jnp.exp(logit) * mk1
        den = dot(seg50, a) + EPS
        wsum = dot(seg50, a * x_ia)
        h_iI = relu(dot(wsum / den, aiWt_) + aib_)

        x1 = dot(relu(dot(eq[...], gvA_) + dot(oq[...], gvB_) + gvb1_),
                 gvW2t_) + gvb2_
        x2 = dot(relu(dot(eer[...], gvA_) + dot(oer[...], gvB_) + gvb1_),
                 gvW2t_) + gvb2_
        nproj = dot(nbr_, uiB_)
        npb = dot(rep10, nproj)
        t1 = relu(dot(x1, uiA_) + mkE * npb + uib1_)
        l1 = jnp.sum(t1 * uiw2_, axis=1, keepdims=True) + uib2_
        t2 = relu(dot(x2, uiA_) + mkO * npb + uib1_)
        l2 = jnp.sum(t2 * uiw2_, axis=1, keepdims=True) + uib2_
        a1 = jnp.exp(l1) * mkE
        a2 = jnp.exp(l2) * mkO
        den_s = dot(seg10, a1) + dot(seg10, a2) + EPS
        num = dot(seg10, a1 * x1 + a2 * x2)
        h_oI = relu(dot(num / den_s, aiWt_) + aib_)
        bt = relu(dot(h_oI, uuA_r[...]) + dot(nbr_, uuB_r[...]) + uub1_r[...])
        bl = jnp.sum(bt * uuw2_r[...], axis=1, keepdims=True) + uub2_r[...]
        be = jnp.exp(bl) * mku
        den_b = dot(seg20, be) + EPS
        s2 = dot(seg20, be * h_oI) / den_b
        h_iS = relu(dot(s2, anWt_r[...]) + anb_r[...])

        h = relu(dot(h_iI, m1A_r[...]) + dot(h_iS, m1B_r[...]) + mb1_r[...])
        h = relu(dot(h, mW2t_r[...]) + mb2_r[...])
        h = relu(dot(h, mW3t_r[...]) + mb3_r[...])
        out[...] = h

    def fixed(shape):
        return pl.BlockSpec(shape, lambda i: (0,) * len(shape))

    in_specs = [
        pl.BlockSpec((R1, D), lambda i: (i, 0)),
        pl.BlockSpec((R1, D), lambda i: (i, 0)),
        pl.BlockSpec((R2, D), lambda i: (EQ_OFF + i, 0)),
        pl.BlockSpec((R2, D), lambda i: (OQ_OFF + i, 0)),
        pl.BlockSpec((R2, D), lambda i: (EQ_OFF + i, 0)),
        pl.BlockSpec((R2, D), lambda i: (OQ_OFF + i, 0)),
        pl.BlockSpec((BU, D), lambda i: (i, 0)),
        pl.BlockSpec((RN, D), lambda i: (i, 0)),
        pl.BlockSpec((R1, 1), lambda i: (i, 0)),
        pl.BlockSpec((R2, 1), lambda i: (i, 0)),
        pl.BlockSpec((R2, 1), lambda i: (i, 0)),
        pl.BlockSpec((RN, 1), lambda i: (i, 0)),
    ] + [fixed(w.shape) for w in (
        gvA, gvB, gvb1, gvW2t, gvb2, uiA, uiB, uib1, uiw2, uib2,
        aiWt, aib, uuA, uuB, uub1, uuw2, uub2, anWt, anb,
        m1A, m1B, mb1, mW2t, mb2, mW3t, mb3)]

    return pl.pallas_call(
        body,
        grid=(G,),
        in_specs=in_specs,
        out_specs=pl.BlockSpec((BU, D), lambda i: (i, 0)),
        out_shape=jax.ShapeDtypeStruct((B, D), jnp.float32),
    )(item_g, rating_g, item_g, item_g, rating_g, rating_g, pu_g, nbr_g,
      mk1_i, mkE_i, mkO_i,
      mku_i, gvA, gvB, gvb1, gvW2t, gvb2, uiA, uiB, uib1, uiw2, uib2,
      aiWt, aib, uuA, uuB, uub1, uuw2, uub2, anWt, anb,
      m1A, m1B, mb1, mW2t, mb2, mW3t, mb3)


def kernel(uids, u_item_pad, u_user_pad, u_user_item_pad, user_table,
           item_table, rating_table, gv_W1, gv_b1, gv_W2, gv_b2, ui_W1,
           ui_b1, ui_W2, ui_b2, ai_W, ai_b, uu_W1, uu_b1, uu_W2, uu_b2,
           an_W, an_b, m_W1, m_b1, m_W2, m_b2, m_W3, m_b3):
    B, Li, _ = u_item_pad.shape
    _, Nn, Mi, _ = u_user_item_pad.shape
    H = Mi // 2
    i32 = jnp.int32

    uip = u_item_pad.astype(i32)
    uuip = u_user_item_pad.astype(i32)
    item_idx = jnp.concatenate([
        uip[:, :, 0].reshape(-1),
        uuip[:, :, 0::2, 0].reshape(-1), uuip[:, :, 1::2, 0].reshape(-1)])
    rating_idx = jnp.concatenate([
        uip[:, :, 1].reshape(-1),
        uuip[:, :, 0::2, 1].reshape(-1), uuip[:, :, 1::2, 1].reshape(-1)])
    n_raw = item_idx.shape[0]
    n_pad = -n_raw % (NW * CHUNK)
    if n_pad:
        pad = jnp.zeros((n_pad,), i32)
        item_idx = jnp.concatenate([item_idx, pad])
        rating_idx = jnp.concatenate([rating_idx, pad])
    nbr_idx = u_user_pad.astype(i32).reshape(-1)
    uid_idx = uids.astype(i32)

    item_g3, rating_g3, pu_g, nbr_g3 = _sc_gather(
        item_table, rating_table, user_table, item_idx, rating_idx,
        uid_idx, nbr_idx)
    item_g = item_g3.reshape(-1, D)
    rating_g = rating_g3.reshape(-1, D)
    nbr_g = nbr_g3.reshape(-1, D)

    mk1_i = uip[:, :, 0].reshape(-1, 1)
    mkE_i = uuip[:, :, :H, 0].reshape(-1, 1)
    mkO_i = uuip[:, :, H:, 0].reshape(-1, 1)
    mku_i = u_user_pad.astype(i32).reshape(-1, 1)

    row = lambda b: b.reshape(1, -1)
    return _tc_compute(
        8, B, Li, Nn, H,
        item_g, rating_g, pu_g, nbr_g, mk1_i, mkE_i, mkO_i, mku_i,
        gv_W1[:, :D].T, gv_W1[:, D:].T, row(gv_b1), gv_W2.T, row(gv_b2),
        ui_W1[:, :D].T, ui_W1[:, D:].T, row(ui_b1), ui_W2, row(ui_b2),
        ai_W.T, row(ai_b),
        uu_W1[:, :D].T, uu_W1[:, D:].T, row(uu_b1), uu_W2, row(uu_b2),
        an_W.T, row(an_b),
        m_W1[:, :D].T, m_W1[:, D:].T, row(m_b1), m_W2.T, row(m_b2),
        m_W3.T, row(m_b3))


# speedup vs baseline: 3.3993x; 1.1119x over previous
import functools

import jax
import jax.numpy as jnp
from jax import lax
from jax.experimental import pallas as pl
from jax.experimental.pallas import tpu as pltpu
from jax.experimental.pallas import tpu_sc as plsc

D = 64
EPS = 1e-10
NW = 32
CHUNK = 128
FIRE = 7


def _sc_gather(item_table, rating_table, user_table, item_idx, rating_idx,
               uid_idx, nbr_idx):
    item_chunks = item_idx.shape[0] // (NW * CHUNK)
    nbr_chunks = nbr_idx.shape[0] // (NW * CHUNK)
    uid_per = uid_idx.shape[0] // NW

    mesh = plsc.VectorSubcoreMesh(core_axis_name="c", subcore_axis_name="s")

    @functools.partial(
        pl.kernel,
        out_type=[
            jax.ShapeDtypeStruct((item_chunks * NW, CHUNK, D), jnp.float32),
            jax.ShapeDtypeStruct((item_chunks * NW, CHUNK, D), jnp.float32),
            jax.ShapeDtypeStruct((uid_idx.shape[0], D), jnp.float32),
            jax.ShapeDtypeStruct((nbr_chunks * NW, CHUNK, D), jnp.float32),
        ],
        mesh=mesh,
        scratch_types=[
            pltpu.VMEM((item_chunks * CHUNK,), jnp.int32),
            pltpu.VMEM((FIRE, CHUNK, D), jnp.float32),
            pltpu.VMEM((uid_per,), jnp.int32),
            pltpu.VMEM((uid_per, D), jnp.float32),
            pltpu.SemaphoreType.DMA,
        ],
        compiler_params=pltpu.CompilerParams(use_tc_tiling_on_sc=False),
    )
    def gather_k(item_t, rating_t, user_t, item_i, rating_i, uid_i, nbr_i,
                 item_o, rating_o, uid_o, nbr_o,
                 idx_v, rows_v, uidx_v, urows_v, sem):
        wid = lax.axis_index("s") * 2 + lax.axis_index("c")

        def job(tab, idx_h, out_h, n_chunks):
            base = wid * n_chunks
            per_tile = n_chunks * CHUNK
            pltpu.sync_copy(idx_h.at[pl.ds(wid * per_tile, per_tile)],
                            idx_v.at[pl.ds(0, per_tile)])
            n_groups = n_chunks // FIRE
            rem = n_chunks - n_groups * FIRE

            def fire_drain(g, k):
                cps = [
                    pltpu.make_async_copy(
                        tab.at[idx_v.at[pl.ds((g * FIRE + j) * CHUNK, CHUNK)]],
                        rows_v.at[j], sem)
                    for j in range(k)
                ]
                for c in cps:
                    c.start()
                for c in cps:
                    c.wait()
                pltpu.sync_copy(rows_v.at[pl.ds(0, k)],
                                out_h.at[pl.ds(base + g * FIRE, k)])

            def body(g, carry):
                fire_drain(g, FIRE)
                return carry

            lax.fori_loop(0, n_groups, body, 0)
            if rem:
                fire_drain(n_groups, rem)

        job(item_t, item_i, item_o, item_chunks)
        job(rating_t, rating_i, rating_o, item_chunks)
        job(user_t, nbr_i, nbr_o, nbr_chunks)
        pltpu.sync_copy(uid_i.at[pl.ds(wid * uid_per, uid_per)], uidx_v)
        pltpu.async_copy(user_t.at[uidx_v], urows_v, sem).wait()
        pltpu.sync_copy(urows_v, uid_o.at[pl.ds(wid * uid_per, uid_per)])

    return gather_k(item_table, rating_table, user_table, item_idx,
                    rating_idx, uid_idx, nbr_idx)


def _seg_mat(per, rows, cols, dtype):
    c = lax.broadcasted_iota(jnp.int32, (rows, cols), 1)
    r = lax.broadcasted_iota(jnp.int32, (rows, cols), 0)
    return (c // per == r).astype(dtype)


def _rep_mat(per, rows, cols, dtype):
    c = lax.broadcasted_iota(jnp.int32, (rows, cols), 1)
    r = lax.broadcasted_iota(jnp.int32, (rows, cols), 0)
    return (r // per == c).astype(dtype)


def _tc_compute(BU, B, Li, Nn, H,
                item_g, rating_g, pu_g, nbr_g, mk1_i, mkE_i, mkO_i, mku_i,
                gvW1t, gvb1, gvW2t, gvb2, uiW1t, uib1, uiw2, uib2,
                aiWt, aib, uuW1t, uub1, uuw2, uub2, anWt, anb,
                mW1t, mb1, mW2t, mb2, mW3t, mb3):
    G = B // BU
    R1 = BU * Li
    R2 = BU * Nn * H
    RN = BU * Nn
    RT = R1 + 2 * R2
    EQ_OFF = (B * Li) // R2
    OQ_OFF = EQ_OFF + (B * Nn * H) // R2
    f32 = jnp.float32

    def dot(a, b):
        return lax.dot_general(a, b, (((1,), (0,)), ((), ())),
                               preferred_element_type=f32)

    def body(qa, ra, eq, oq, eer, oer, pu, nbr, mk1i, mkEi, mkOi, mkui,
             gvW1t_r, gvb1_r, gvW2t_r, gvb2_r, uiW1t_r, uib1_r,
             uiw2_r, uib2_r, aiWt_r, aib_r, uuW1t_r, uub1_r, uuw2_r,
             uub2_r, anWt_r, anb_r, mW1t_r, mb1_r, mW2t_r, mb2_r,
             mW3t_r, mb3_r, out):
        relu = lambda x: jnp.maximum(x, 0.0)
        cat1 = lambda xs: jnp.concatenate(xs, axis=1)
        cat0 = lambda xs: jnp.concatenate(xs, axis=0)

        seg50 = _seg_mat(Li, BU, R1, f32)
        rep50 = _rep_mat(Li, R1, BU, f32)
        seg10 = _seg_mat(H, RN, R2, f32)
        rep10 = _rep_mat(H, R2, RN, f32)
        seg20 = _seg_mat(Nn, BU, RN, f32)

        mk1 = (mk1i[...] > 0).astype(f32)
        mkE = (mkEi[...] > 0).astype(f32)
        mkO = (mkOi[...] > 0).astype(f32)
        mku = (mkui[...] > 0).astype(f32)
        pu_ = pu[...]
        nbr_ = nbr[...]

        X = cat0([cat1([qa[...], ra[...]]), cat1([eq[...], oq[...]]),
                  cat1([eer[...], oer[...]])])
        xh = relu(dot(X, gvW1t_r[...]) + gvb1_r[...])
        xall = dot(xh, gvW2t_r[...]) + gvb2_r[...]

        P = cat0([mk1 * dot(rep50, pu_),
                  mkE * dot(rep10, nbr_), mkO * dot(rep10, nbr_)])
        t = relu(dot(cat1([xall, P]), uiW1t_r[...]) + uib1_r[...])
        logit = jnp.sum(t * uiw2_r[...], axis=1, keepdims=True) + uib2_r[...]
        M = cat0([mk1, mkE, mkO])
        a = jnp.exp(logit) * M
        ax = a * xall

        den1 = dot(seg50, a[:R1]) + EPS
        wsum = dot(seg50, ax[:R1])
        den_s = dot(seg10, a[R1:R1 + R2] + a[R1 + R2:]) + EPS
        num = dot(seg10, ax[R1:R1 + R2] + ax[R1 + R2:])

        agg = cat0([wsum / den1, num / den_s])
        hagg = relu(dot(agg, aiWt_r[...]) + aib_r[...])
        h_iI = hagg[:BU]
        h_oI = hagg[BU:]

        bt = relu(dot(cat1([h_oI, nbr_]), uuW1t_r[...]) + uub1_r[...])
        bl = jnp.sum(bt * uuw2_r[...], axis=1, keepdims=True) + uub2_r[...]
        be = jnp.exp(bl) * mku
        den_b = dot(seg20, be) + EPS
        s2 = dot(seg20, be * h_oI) / den_b
        h_iS = relu(dot(s2, anWt_r[...]) + anb_r[...])

        h = relu(dot(cat1([h_iI, h_iS]), mW1t_r[...]) + mb1_r[...])
        h = relu(dot(h, mW2t_r[...]) + mb2_r[...])
        h = relu(dot(h, mW3t_r[...]) + mb3_r[...])
        out[...] = h

    def fixed(shape):
        return pl.BlockSpec(shape, lambda i: (0,) * len(shape))

    in_specs = [
        pl.BlockSpec((R1, D), lambda i: (i, 0)),
        pl.BlockSpec((R1, D), lambda i: (i, 0)),
        pl.BlockSpec((R2, D), lambda i: (EQ_OFF + i, 0)),
        pl.BlockSpec((R2, D), lambda i: (OQ_OFF + i, 0)),
        pl.BlockSpec((R2, D), lambda i: (EQ_OFF + i, 0)),
        pl.BlockSpec((R2, D), lambda i: (OQ_OFF + i, 0)),
        pl.BlockSpec((BU, D), lambda i: (i, 0)),
        pl.BlockSpec((RN, D), lambda i: (i, 0)),
        pl.BlockSpec((R1, 1), lambda i: (i, 0)),
        pl.BlockSpec((R2, 1), lambda i: (i, 0)),
        pl.BlockSpec((R2, 1), lambda i: (i, 0)),
        pl.BlockSpec((RN, 1), lambda i: (i, 0)),
    ] + [fixed(w.shape) for w in (
        gvW1t, gvb1, gvW2t, gvb2, uiW1t, uib1, uiw2, uib2,
        aiWt, aib, uuW1t, uub1, uuw2, uub2, anWt, anb,
        mW1t, mb1, mW2t, mb2, mW3t, mb3)]

    return pl.pallas_call(
        body,
        grid=(G,),
        in_specs=in_specs,
        out_specs=pl.BlockSpec((BU, D), lambda i: (i, 0)),
        out_shape=jax.ShapeDtypeStruct((B, D), jnp.float32),
    )(item_g, rating_g, item_g, item_g, rating_g, rating_g, pu_g, nbr_g,
      mk1_i, mkE_i, mkO_i, mku_i,
      gvW1t, gvb1, gvW2t, gvb2, uiW1t, uib1, uiw2, uib2,
      aiWt, aib, uuW1t, uub1, uuw2, uub2, anWt, anb,
      mW1t, mb1, mW2t, mb2, mW3t, mb3)


def kernel(uids, u_item_pad, u_user_pad, u_user_item_pad, user_table,
           item_table, rating_table, gv_W1, gv_b1, gv_W2, gv_b2, ui_W1,
           ui_b1, ui_W2, ui_b2, ai_W, ai_b, uu_W1, uu_b1, uu_W2, uu_b2,
           an_W, an_b, m_W1, m_b1, m_W2, m_b2, m_W3, m_b3):
    B, Li, _ = u_item_pad.shape
    _, Nn, Mi, _ = u_user_item_pad.shape
    H = Mi // 2
    i32 = jnp.int32

    uip = u_item_pad.astype(i32)
    uuip = u_user_item_pad.astype(i32)
    item_idx = jnp.concatenate([
        uip[:, :, 0].reshape(-1),
        uuip[:, :, 0::2, 0].reshape(-1), uuip[:, :, 1::2, 0].reshape(-1)])
    rating_idx = jnp.concatenate([
        uip[:, :, 1].reshape(-1),
        uuip[:, :, 0::2, 1].reshape(-1), uuip[:, :, 1::2, 1].reshape(-1)])
    n_raw = item_idx.shape[0]
    n_pad = -n_raw % (NW * CHUNK)
    if n_pad:
        pad = jnp.zeros((n_pad,), i32)
        item_idx = jnp.concatenate([item_idx, pad])
        rating_idx = jnp.concatenate([rating_idx, pad])
    nbr_idx = u_user_pad.astype(i32).reshape(-1)
    uid_idx = uids.astype(i32)

    item_g3, rating_g3, pu_g, nbr_g3 = _sc_gather(
        item_table, rating_table, user_table, item_idx, rating_idx,
        uid_idx, nbr_idx)
    item_g = item_g3.reshape(-1, D)
    rating_g = rating_g3.reshape(-1, D)
    nbr_g = nbr_g3.reshape(-1, D)

    mk1_i = uip[:, :, 0].reshape(-1, 1)
    mkE_i = uuip[:, :, :H, 0].reshape(-1, 1)
    mkO_i = uuip[:, :, H:, 0].reshape(-1, 1)
    mku_i = u_user_pad.astype(i32).reshape(-1, 1)

    row = lambda b: b.reshape(1, -1)
    return _tc_compute(
        8, B, Li, Nn, H,
        item_g, rating_g, pu_g, nbr_g, mk1_i, mkE_i, mkO_i, mku_i,
        gv_W1.T, row(gv_b1), gv_W2.T, row(gv_b2),
        ui_W1.T, row(ui_b1), ui_W2, row(ui_b2),
        ai_W.T, row(ai_b),
        uu_W1.T, row(uu_b1), uu_W2, row(uu_b2),
        an_W.T, row(an_b),
        m_W1.T, row(m_b1), m_W2.T, row(m_b2),
        m_W3.T, row(m_b3))
